# baseline (speedup 1.0000x reference)
import functools

import jax
import jax.numpy as jnp
from jax.experimental import pallas as pl
from jax.experimental.pallas import tpu as pltpu

_K = 5
_LOCAL_BINS = 10
_TOTAL_BINS = _LOCAL_BINS * 2 * 2 * 2

_INF = float("inf")
_BIG_I = 2**31 - 1


def _knn_body(rowb_ref, colb_ref, idx_ref, bval, bidx, *, ncol, blk_c):
    j = pl.program_id(1)

    @pl.when(j == 0)
    def _init():
        bval[...] = jnp.full(bval.shape, _INF, jnp.float32)
        bidx[...] = jnp.full(bidx.shape, _BIG_I, jnp.int32)

    def _rb(v):
        return v.astype(jnp.bfloat16).astype(jnp.float32)

    xr = _rb(rowb_ref[:, 0:1])
    yr = _rb(rowb_ref[:, 1:2])
    zr = _rb(rowb_ref[:, 2:3])
    pr = rowb_ref[:, 3:4]
    xc = _rb(colb_ref[0:1, :])
    yc = _rb(colb_ref[1:2, :])
    zc = _rb(colb_ref[2:3, :])
    pc = colb_ref[3:4, :]
    d = (pr + pc) - 2.0 * (xr * xc + yr * yc + zr * zc)

    cols = jax.lax.broadcasted_iota(jnp.int32, (1, blk_c), 1) + j * blk_c

    bv = []
    bi = []
    for _ in range(_K):
        m = jnp.min(d, axis=1, keepdims=True)
        am = jnp.min(jnp.where(d == m, cols, _BIG_I), axis=1, keepdims=True)
        bv.append(m)
        bi.append(am)
        d = jnp.where(cols == am, jnp.inf, d)

    cv = jnp.concatenate([bval[...]] + bv, axis=1)
    ci = jnp.concatenate([bidx[...]] + bi, axis=1)
    nv = []
    ni = []
    for _ in range(_K):
        m = jnp.min(cv, axis=1, keepdims=True)
        am = jnp.min(jnp.where(cv == m, ci, _BIG_I), axis=1, keepdims=True)
        nv.append(m)
        ni.append(am)
        cv = jnp.where(ci == am, jnp.inf, cv)
    bval[...] = jnp.concatenate(nv, axis=1)
    bidx[...] = jnp.concatenate(ni, axis=1)

    @pl.when(j == ncol - 1)
    def _flush():
        idx_ref[...] = bidx[...]


def _knn_pallas(points, interpret=False, blk_r=256, blk_c=8192):
    n = points.shape[0]
    blk_r = min(blk_r, n)
    blk_c = min(blk_c, n)
    nrow = n // blk_r
    ncol = n // blk_c
    pn = jnp.sum(points * points, axis=-1)
    rowb = jnp.concatenate([points, pn[:, None]], axis=1)
    colb = jnp.concatenate([points.T, pn[None, :]], axis=0)
    body = functools.partial(_knn_body, ncol=ncol, blk_c=blk_c)
    return pl.pallas_call(
        body,
        grid=(nrow, ncol),
        in_specs=[
            pl.BlockSpec((blk_r, 4), lambda i, j: (i, 0)),
            pl.BlockSpec((4, blk_c), lambda i, j: (0, j)),
        ],
        out_specs=pl.BlockSpec((blk_r, _K), lambda i, j: (i, 0)),
        out_shape=jax.ShapeDtypeStruct((n, _K), jnp.int32),
        scratch_shapes=[
            pltpu.VMEM((blk_r, _K), jnp.float32),
            pltpu.VMEM((blk_r, _K), jnp.int32),
        ],
        interpret=interpret,
    )(rowb, colb)


def kernel(points, batch):
    nbh_idx = _knn_pallas(points)
    nbh = points[nbh_idx]
    mean = jnp.mean(nbh, axis=1, keepdims=True)
    diffs = nbh - mean
    cov = jnp.einsum('nki,nkj->nij', diffs, diffs) / nbh.shape[1]
    _, lrfs = jnp.linalg.eigh(cov)
    return lrfs


def _kernel_full(points, batch):
    nbh_idx = _knn_pallas(points)
    nbh = points[nbh_idx]
    mean = jnp.mean(nbh, axis=1, keepdims=True)
    diffs = nbh - mean
    cov = jnp.einsum('nki,nkj->nij', diffs, diffs) / nbh.shape[1]
    _, lrfs = jnp.linalg.eigh(cov)
    normals = lrfs[:, :, 0]
    nbh_proj = jnp.einsum('nki,nij->nkj', nbh, lrfs)
    upper_x = (nbh_proj[:, :, 0] >= 0).astype(jnp.int32)
    upper_y = (nbh_proj[:, :, 1] >= 0).astype(jnp.int32)
    upper_z = (nbh_proj[:, :, 2] >= 0).astype(jnp.int32)
    spatial_id = (upper_x << 2) + (upper_y << 1) + upper_z
    cos = jnp.sum(normals[:, None, :] * normals[nbh_idx], axis=-1)
    normal_id = jnp.floor(_LOCAL_BINS * (cos + 1.0) / 2.0)
    normal_id = jnp.clip(normal_id, 0, _LOCAL_BINS - 1)
    bin_id = spatial_id.astype(jnp.float32) * _LOCAL_BINS + normal_id
    bin_idx = bin_id.astype(jnp.int32)
    rows = jnp.broadcast_to(jnp.arange(points.shape[0])[:, None], bin_idx.shape)
    shot = jnp.zeros((points.shape[0], _TOTAL_BINS), jnp.float32).at[rows, bin_idx].add(1.0)
    return shot


# speedup vs baseline: 10.2528x; 10.2528x over previous
import functools

import jax
import jax.numpy as jnp
from jax.experimental import pallas as pl
from jax.experimental.pallas import tpu as pltpu

_K = 5
_LOCAL_BINS = 10
_TOTAL_BINS = _LOCAL_BINS * 2 * 2 * 2

_INF = float("inf")
_BIG_I = 2**31 - 1


def _knn_body(rowb_ref, colb_ref, idx_ref, bval, bidx, *, ncol, blk_c):
    j = pl.program_id(1)

    @pl.when(j == 0)
    def _init():
        bval[...] = jnp.full(bval.shape, _INF, jnp.float32)
        bidx[...] = jnp.full(bidx.shape, _BIG_I, jnp.int32)

    def _rb(v):
        return v.astype(jnp.bfloat16).astype(jnp.float32)

    xr = _rb(rowb_ref[:, 0:1])
    yr = _rb(rowb_ref[:, 1:2])
    zr = _rb(rowb_ref[:, 2:3])
    pr = rowb_ref[:, 3:4]
    xc = _rb(colb_ref[0:1, :])
    yc = _rb(colb_ref[1:2, :])
    zc = _rb(colb_ref[2:3, :])
    pc = colb_ref[3:4, :]
    d = (pr + pc) - 2.0 * (xr * xc + yr * yc + zr * zc)

    cols = jax.lax.broadcasted_iota(jnp.int32, (1, blk_c), 1) + j * blk_c

    bv = []
    bi = []
    for _ in range(_K):
        m = jnp.min(d, axis=1, keepdims=True)
        am = jnp.min(jnp.where(d == m, cols, _BIG_I), axis=1, keepdims=True)
        bv.append(m)
        bi.append(am)
        d = jnp.where(cols == am, jnp.inf, d)

    cv = jnp.concatenate([bval[...]] + bv, axis=1)
    ci = jnp.concatenate([bidx[...]] + bi, axis=1)
    nv = []
    ni = []
    for _ in range(_K):
        m = jnp.min(cv, axis=1, keepdims=True)
        am = jnp.min(jnp.where(cv == m, ci, _BIG_I), axis=1, keepdims=True)
        nv.append(m)
        ni.append(am)
        cv = jnp.where(ci == am, jnp.inf, cv)
    bval[...] = jnp.concatenate(nv, axis=1)
    bidx[...] = jnp.concatenate(ni, axis=1)

    @pl.when(j == ncol - 1)
    def _flush():
        idx_ref[...] = bidx[...]


def _knn_pallas(points, interpret=False, blk_r=256, blk_c=8192):
    n = points.shape[0]
    blk_r = min(blk_r, n)
    blk_c = min(blk_c, n)
    nrow = n // blk_r
    ncol = n // blk_c
    pn = jnp.sum(points * points, axis=-1)
    rowb = jnp.concatenate([points, pn[:, None]], axis=1)
    colb = jnp.concatenate([points.T, pn[None, :]], axis=0)
    body = functools.partial(_knn_body, ncol=ncol, blk_c=blk_c)
    return pl.pallas_call(
        body,
        grid=(nrow, ncol),
        in_specs=[
            pl.BlockSpec((blk_r, 4), lambda i, j: (i, 0)),
            pl.BlockSpec((4, blk_c), lambda i, j: (0, j)),
        ],
        out_specs=pl.BlockSpec((blk_r, _K), lambda i, j: (i, 0)),
        out_shape=jax.ShapeDtypeStruct((n, _K), jnp.int32),
        scratch_shapes=[
            pltpu.VMEM((blk_r, _K), jnp.float32),
            pltpu.VMEM((blk_r, _K), jnp.int32),
        ],
        interpret=interpret,
    )(rowb, colb)


_SWEEPS = 6
_JACOBI_PAIRS = [(0, 2), (2, 1), (0, 1)]


def _eigh3_body(cin_ref, vout_ref):
    A = [[None] * 3 for _ in range(3)]
    A[0][0] = cin_ref[0]
    A[0][1] = A[1][0] = cin_ref[1]
    A[0][2] = A[2][0] = cin_ref[2]
    A[1][1] = cin_ref[3]
    A[1][2] = A[2][1] = cin_ref[4]
    A[2][2] = cin_ref[5]
    one = jnp.ones_like(A[0][0])
    zero = jnp.zeros_like(A[0][0])
    V = [[one if i == j else zero for j in range(3)] for i in range(3)]

    for _ in range(_SWEEPS):
        for (p, q) in _JACOBI_PAIRS:
            r = 3 - p - q
            wpp, wqq, wpq = A[p][p], A[q][q], A[p][q]
            tau = (wqq - wpp) / (2.0 * wpq)
            t = jnp.sign(tau) / (jnp.abs(tau) + jnp.sqrt(1.0 + tau * tau))
            c = jax.lax.rsqrt(1.0 + t * t)
            s = t * c
            c = jnp.where(wpq == 0.0, 1.0, c)
            s = jnp.where(wpq == 0.0, 0.0, s)
            cc, ss, cs = c * c, s * s, c * s
            apr, aqr = A[p][r], A[q][r]
            new_pp = cc * wpp - 2.0 * cs * wpq + ss * wqq
            new_qq = ss * wpp + 2.0 * cs * wpq + cc * wqq
            new_pq = (cc - ss) * wpq + cs * (wpp - wqq)
            new_pr = c * apr - s * aqr
            new_qr = s * apr + c * aqr
            A[p][p] = new_pp
            A[q][q] = new_qq
            A[p][q] = A[q][p] = new_pq
            A[p][r] = A[r][p] = new_pr
            A[q][r] = A[r][q] = new_qr
            for i in range(3):
                vip, viq = V[i][p], V[i][q]
                V[i][p] = c * vip - s * viq
                V[i][q] = s * vip + c * viq

    e0, e1, e2 = A[0][0], A[1][1], A[2][2]
    evs = [e0, e1, e2]
    m01 = jnp.where(e1 < e0, 1, 0)
    em01 = jnp.where(e1 < e0, e1, e0)
    first = jnp.where(e2 < em01, 2, m01)
    m21 = jnp.where(e1 > e2, 1, 2)
    em21 = jnp.where(e1 > e2, e1, e2)
    last = jnp.where(e0 > em21, 0, m21)
    middle = 3 - first - last
    for j, ordj in enumerate((first, middle, last)):
        for i in range(3):
            col = jnp.where(ordj == 0, V[i][0],
                            jnp.where(ordj == 1, V[i][1], V[i][2]))
            vout_ref[3 * i + j] = col


def _eigh3_pallas(cov, interpret=False):
    n = cov.shape[0]
    rows = n // 128
    planes = jnp.stack([
        cov[:, 0, 0], cov[:, 0, 1], cov[:, 0, 2],
        cov[:, 1, 1], cov[:, 1, 2], cov[:, 2, 2],
    ]).reshape(6, rows, 128)
    vp = pl.pallas_call(
        _eigh3_body,
        out_shape=jax.ShapeDtypeStruct((9, rows, 128), jnp.float32),
        interpret=interpret,
    )(planes)
    return vp.reshape(9, n).T.reshape(n, 3, 3)


def kernel(points, batch):
    nbh_idx = _knn_pallas(points)
    nbh = points[nbh_idx]
    mean = jnp.mean(nbh, axis=1, keepdims=True)
    diffs = nbh - mean
    cov = jnp.einsum('nki,nkj->nij', diffs, diffs) / nbh.shape[1]
    lrfs = _eigh3_pallas(cov)
    normals = lrfs[:, :, 0]
    nbh_proj = jnp.einsum('nki,nij->nkj', nbh, lrfs)
    upper_x = (nbh_proj[:, :, 0] >= 0).astype(jnp.int32)
    upper_y = (nbh_proj[:, :, 1] >= 0).astype(jnp.int32)
    upper_z = (nbh_proj[:, :, 2] >= 0).astype(jnp.int32)
    spatial_id = (upper_x << 2) + (upper_y << 1) + upper_z
    cos = jnp.sum(normals[:, None, :] * normals[nbh_idx], axis=-1)
    normal_id = jnp.floor(_LOCAL_BINS * (cos + 1.0) / 2.0)
    normal_id = jnp.clip(normal_id, 0, _LOCAL_BINS - 1)
    bin_id = spatial_id.astype(jnp.float32) * _LOCAL_BINS + normal_id
    bin_idx = bin_id.astype(jnp.int32)
    rows = jnp.broadcast_to(jnp.arange(points.shape[0])[:, None], bin_idx.shape)
    shot = jnp.zeros((points.shape[0], _TOTAL_BINS), jnp.float32).at[rows, bin_idx].add(1.0)
    return shot


def _kernel_old(points, batch):
    nbh_idx = _knn_pallas(points)
    nbh = points[nbh_idx]
    mean = jnp.mean(nbh, axis=1, keepdims=True)
    diffs = nbh - mean
    cov = jnp.einsum('nki,nkj->nij', diffs, diffs) / nbh.shape[1]
    _, lrfs = jnp.linalg.eigh(cov)
    normals = lrfs[:, :, 0]
    nbh_proj = jnp.einsum('nki,nij->nkj', nbh, lrfs)
    upper_x = (nbh_proj[:, :, 0] >= 0).astype(jnp.int32)
    upper_y = (nbh_proj[:, :, 1] >= 0).astype(jnp.int32)
    upper_z = (nbh_proj[:, :, 2] >= 0).astype(jnp.int32)
    spatial_id = (upper_x << 2) + (upper_y << 1) + upper_z
    cos = jnp.sum(normals[:, None, :] * normals[nbh_idx], axis=-1)
    normal_id = jnp.floor(_LOCAL_BINS * (cos + 1.0) / 2.0)
    normal_id = jnp.clip(normal_id, 0, _LOCAL_BINS - 1)
    bin_id = spatial_id.astype(jnp.float32) * _LOCAL_BINS + normal_id
    bin_idx = bin_id.astype(jnp.int32)
    rows = jnp.broadcast_to(jnp.arange(points.shape[0])[:, None], bin_idx.shape)
    shot = jnp.zeros((points.shape[0], _TOTAL_BINS), jnp.float32).at[rows, bin_idx].add(1.0)
    return shot


# speedup vs baseline: 14.2839x; 1.3932x over previous
import functools

import jax
import jax.numpy as jnp
from jax import lax
from jax.experimental import pallas as pl
from jax.experimental.pallas import tpu as pltpu
from jax.experimental.pallas import tpu_sc as plsc

_K = 5
_LOCAL_BINS = 10
_TOTAL_BINS = _LOCAL_BINS * 2 * 2 * 2

_INF = float("inf")
_BIG_I = 2**31 - 1


def _knn_body(rowb_ref, colb_ref, idx_ref, bval, bidx, *, ncol, blk_c):
    j = pl.program_id(1)

    @pl.when(j == 0)
    def _init():
        bval[...] = jnp.full(bval.shape, _INF, jnp.float32)
        bidx[...] = jnp.full(bidx.shape, _BIG_I, jnp.int32)

    def _rb(v):
        return v.astype(jnp.bfloat16).astype(jnp.float32)

    xr = _rb(rowb_ref[:, 0:1])
    yr = _rb(rowb_ref[:, 1:2])
    zr = _rb(rowb_ref[:, 2:3])
    pr = rowb_ref[:, 3:4]
    xc = _rb(colb_ref[0:1, :])
    yc = _rb(colb_ref[1:2, :])
    zc = _rb(colb_ref[2:3, :])
    pc = colb_ref[3:4, :]
    d = (pr + pc) - 2.0 * (xr * xc + yr * yc + zr * zc)

    cols = jax.lax.broadcasted_iota(jnp.int32, (1, blk_c), 1) + j * blk_c

    bv = []
    bi = []
    for _ in range(_K):
        m = jnp.min(d, axis=1, keepdims=True)
        am = jnp.min(jnp.where(d == m, cols, _BIG_I), axis=1, keepdims=True)
        bv.append(m)
        bi.append(am)
        d = jnp.where(cols == am, jnp.inf, d)

    cv = jnp.concatenate([bval[...]] + bv, axis=1)
    ci = jnp.concatenate([bidx[...]] + bi, axis=1)
    nv = []
    ni = []
    for _ in range(_K):
        m = jnp.min(cv, axis=1, keepdims=True)
        am = jnp.min(jnp.where(cv == m, ci, _BIG_I), axis=1, keepdims=True)
        nv.append(m)
        ni.append(am)
        cv = jnp.where(ci == am, jnp.inf, cv)
    bval[...] = jnp.concatenate(nv, axis=1)
    bidx[...] = jnp.concatenate(ni, axis=1)

    @pl.when(j == ncol - 1)
    def _flush():
        idx_ref[...] = bidx[...]


def _knn_pallas(points, interpret=False, blk_r=256, blk_c=8192):
    n = points.shape[0]
    blk_r = min(blk_r, n)
    blk_c = min(blk_c, n)
    nrow = n // blk_r
    ncol = n // blk_c
    pn = jnp.sum(points * points, axis=-1)
    rowb = jnp.concatenate([points, pn[:, None]], axis=1)
    colb = jnp.concatenate([points.T, pn[None, :]], axis=0)
    body = functools.partial(_knn_body, ncol=ncol, blk_c=blk_c)
    return pl.pallas_call(
        body,
        grid=(nrow, ncol),
        in_specs=[
            pl.BlockSpec((blk_r, 4), lambda i, j: (i, 0)),
            pl.BlockSpec((4, blk_c), lambda i, j: (0, j)),
        ],
        out_specs=pl.BlockSpec((blk_r, _K), lambda i, j: (i, 0)),
        out_shape=jax.ShapeDtypeStruct((n, _K), jnp.int32),
        scratch_shapes=[
            pltpu.VMEM((blk_r, _K), jnp.float32),
            pltpu.VMEM((blk_r, _K), jnp.int32),
        ],
        interpret=interpret,
    )(rowb, colb)


_SWEEPS = 6
_JACOBI_PAIRS = [(0, 2), (2, 1), (0, 1)]


def _eigh3_body(cin_ref, vout_ref):
    A = [[None] * 3 for _ in range(3)]
    A[0][0] = cin_ref[0]
    A[0][1] = A[1][0] = cin_ref[1]
    A[0][2] = A[2][0] = cin_ref[2]
    A[1][1] = cin_ref[3]
    A[1][2] = A[2][1] = cin_ref[4]
    A[2][2] = cin_ref[5]
    one = jnp.ones_like(A[0][0])
    zero = jnp.zeros_like(A[0][0])
    V = [[one if i == j else zero for j in range(3)] for i in range(3)]

    for _ in range(_SWEEPS):
        for (p, q) in _JACOBI_PAIRS:
            r = 3 - p - q
            wpp, wqq, wpq = A[p][p], A[q][q], A[p][q]
            tau = (wqq - wpp) / (2.0 * wpq)
            t = jnp.sign(tau) / (jnp.abs(tau) + jnp.sqrt(1.0 + tau * tau))
            c = jax.lax.rsqrt(1.0 + t * t)
            s = t * c
            c = jnp.where(wpq == 0.0, 1.0, c)
            s = jnp.where(wpq == 0.0, 0.0, s)
            cc, ss, cs = c * c, s * s, c * s
            apr, aqr = A[p][r], A[q][r]
            new_pp = cc * wpp - 2.0 * cs * wpq + ss * wqq
            new_qq = ss * wpp + 2.0 * cs * wpq + cc * wqq
            new_pq = (cc - ss) * wpq + cs * (wpp - wqq)
            new_pr = c * apr - s * aqr
            new_qr = s * apr + c * aqr
            A[p][p] = new_pp
            A[q][q] = new_qq
            A[p][q] = A[q][p] = new_pq
            A[p][r] = A[r][p] = new_pr
            A[q][r] = A[r][q] = new_qr
            for i in range(3):
                vip, viq = V[i][p], V[i][q]
                V[i][p] = c * vip - s * viq
                V[i][q] = s * vip + c * viq

    e0, e1, e2 = A[0][0], A[1][1], A[2][2]
    evs = [e0, e1, e2]
    m01 = jnp.where(e1 < e0, 1, 0)
    em01 = jnp.where(e1 < e0, e1, e0)
    first = jnp.where(e2 < em01, 2, m01)
    m21 = jnp.where(e1 > e2, 1, 2)
    em21 = jnp.where(e1 > e2, e1, e2)
    last = jnp.where(e0 > em21, 0, m21)
    middle = 3 - first - last
    for j, ordj in enumerate((first, middle, last)):
        for i in range(3):
            col = jnp.where(ordj == 0, V[i][0],
                            jnp.where(ordj == 1, V[i][1], V[i][2]))
            vout_ref[3 * i + j] = col


def _eigh3_planes_pallas(planes, interpret=False):
    _, rows, _ = planes.shape
    vp = pl.pallas_call(
        _eigh3_body,
        out_shape=jax.ShapeDtypeStruct((9, rows, 128), jnp.float32),
        interpret=interpret,
    )(planes)
    return vp.reshape(9, rows * 128)


def _eigh3_pallas(cov, interpret=False):
    n = cov.shape[0]
    planes = jnp.stack([
        cov[:, 0, 0], cov[:, 0, 1], cov[:, 0, 2],
        cov[:, 1, 1], cov[:, 1, 2], cov[:, 2, 2],
    ]).reshape(6, n // 128, 128)
    return _eigh3_planes_pallas(planes, interpret).T.reshape(n, 3, 3)


def _sc_workers():
    try:
        info = plsc.get_sparse_core_info()
        return info.num_cores, info.num_subcores
    except Exception:
        return 2, 16


def _rb16(v):
    u = plsc.bitcast(v, jnp.int32)
    r = (u + 0x7FFF + ((u >> 16) & 1)) & jnp.int32(-65536)
    return plsc.bitcast(r, jnp.float32)


def _cov_sc(points, idx_sh):
    n = points.shape[0]
    nc, ns = _sc_workers()
    nw = nc * ns
    npts = n // nw
    nchunks = npts // 16
    pts_flat = points.T.reshape(3 * n)
    mesh = plsc.VectorSubcoreMesh(core_axis_name="c", subcore_axis_name="s")

    @functools.partial(
        pl.kernel, mesh=mesh,
        compiler_params=pltpu.CompilerParams(needs_layout_passes=False),
        out_type=(jax.ShapeDtypeStruct((nw * 6 * npts,), jnp.float32),
                  jax.ShapeDtypeStruct((nw * 15 * npts,), jnp.float32)),
        scratch_types=[
            pltpu.VMEM((n,), jnp.float32),
            pltpu.VMEM((n,), jnp.float32),
            pltpu.VMEM((n,), jnp.float32),
            pltpu.VMEM((5 * npts,), jnp.int32),
            pltpu.VMEM((6 * npts,), jnp.float32),
            pltpu.VMEM((15 * npts,), jnp.float32),
        ],
    )
    def k(pts_hbm, idx_hbm, cov_hbm, nbh_hbm, px, py, pz, idx_v, cov_v, nbh_v):
        wid = lax.axis_index("s") * nc + lax.axis_index("c")
        pltpu.sync_copy(pts_hbm.at[pl.ds(0, n)], px)
        pltpu.sync_copy(pts_hbm.at[pl.ds(n, n)], py)
        pltpu.sync_copy(pts_hbm.at[pl.ds(2 * n, n)], pz)
        pltpu.sync_copy(idx_hbm.at[pl.ds(wid * 5 * npts, 5 * npts)], idx_v)

        def chunk(ci, carry):
            o = ci * 16
            xs, ys, zs = [], [], []
            for kk in range(_K):
                idxk = idx_v[pl.ds(kk * npts + o, 16)]
                xk = plsc.load_gather(px, [idxk])
                yk = plsc.load_gather(py, [idxk])
                zk = plsc.load_gather(pz, [idxk])
                nbh_v[pl.ds((3 * kk + 0) * npts + o, 16)] = xk
                nbh_v[pl.ds((3 * kk + 1) * npts + o, 16)] = yk
                nbh_v[pl.ds((3 * kk + 2) * npts + o, 16)] = zk
                xs.append(xk)
                ys.append(yk)
                zs.append(zk)
            mx = (((xs[0] + xs[4]) + xs[2]) + (xs[1] + xs[3])) * 0.2
            my = (((ys[0] + ys[4]) + ys[2]) + (ys[1] + ys[3])) * 0.2
            mz = (((zs[0] + zs[4]) + zs[2]) + (zs[1] + zs[3])) * 0.2
            dx = [_rb16(x - mx) for x in xs]
            dy = [_rb16(y - my) for y in ys]
            dz = [_rb16(z - mz) for z in zs]

            def dot5(a, b):
                p = [a[i] * b[i] for i in range(_K)]
                return (((p[0] + p[1]) + (p[2] + p[3])) + p[4]) * 0.2

            cov_v[pl.ds(0 * npts + o, 16)] = dot5(dx, dx)
            cov_v[pl.ds(1 * npts + o, 16)] = dot5(dx, dy)
            cov_v[pl.ds(2 * npts + o, 16)] = dot5(dx, dz)
            cov_v[pl.ds(3 * npts + o, 16)] = dot5(dy, dy)
            cov_v[pl.ds(4 * npts + o, 16)] = dot5(dy, dz)
            cov_v[pl.ds(5 * npts + o, 16)] = dot5(dz, dz)
            return carry

        lax.fori_loop(0, nchunks, chunk, 0)
        pltpu.sync_copy(cov_v, cov_hbm.at[pl.ds(wid * 6 * npts, 6 * npts)])
        pltpu.sync_copy(nbh_v, nbh_hbm.at[pl.ds(wid * 15 * npts, 15 * npts)])

    return k(pts_flat, idx_sh.reshape(-1))


def _hist_sc(nbh_flat, idx_flat, lrfp):
    nc, ns = _sc_workers()
    nw = nc * ns
    npts = nbh_flat.shape[0] // (15 * nw)
    n = nw * npts
    nchunks = npts // 16
    nrm_flat = jnp.concatenate([lrfp[0], lrfp[3], lrfp[6]], axis=0)
    lrf_flat = lrfp.reshape(9, nw, npts).transpose(1, 0, 2).reshape(-1)
    mesh = plsc.VectorSubcoreMesh(core_axis_name="c", subcore_axis_name="s")

    @functools.partial(
        pl.kernel, mesh=mesh,
        compiler_params=pltpu.CompilerParams(needs_layout_passes=False),
        out_type=jax.ShapeDtypeStruct((n * _TOTAL_BINS,), jnp.float32),
        scratch_types=[
            pltpu.VMEM((n,), jnp.float32),
            pltpu.VMEM((n,), jnp.float32),
            pltpu.VMEM((n,), jnp.float32),
            pltpu.VMEM((5 * npts,), jnp.int32),
            pltpu.VMEM((15 * npts,), jnp.float32),
            pltpu.VMEM((9 * npts,), jnp.float32),
            pltpu.VMEM((16 * _TOTAL_BINS,), jnp.float32),
        ],
    )
    def k(nbh_hbm, idx_hbm, nrm_hbm, lrf_hbm, out_hbm,
          n0, n1, n2, idx_v, nbh_v, lrf_v, hist_v):
        wid = lax.axis_index("s") * nc + lax.axis_index("c")
        pltpu.sync_copy(nrm_hbm.at[pl.ds(0, n)], n0)
        pltpu.sync_copy(nrm_hbm.at[pl.ds(n, n)], n1)
        pltpu.sync_copy(nrm_hbm.at[pl.ds(2 * n, n)], n2)
        pltpu.sync_copy(idx_hbm.at[pl.ds(wid * 5 * npts, 5 * npts)], idx_v)
        pltpu.sync_copy(nbh_hbm.at[pl.ds(wid * 15 * npts, 15 * npts)], nbh_v)
        pltpu.sync_copy(lrf_hbm.at[pl.ds(wid * 9 * npts, 9 * npts)], lrf_v)

        zeros = jnp.zeros((16,), jnp.float32)
        ones = jnp.ones((16,), jnp.float32)
        lane = lax.iota(jnp.int32, 16)

        def chunk(ci, carry):
            o = ci * 16
            for b in range(_TOTAL_BINS):
                hist_v[pl.ds(b * 16, 16)] = zeros
            l = [lrf_v[pl.ds(p * npts + o, 16)] for p in range(9)]
            n0s = l[0]
            n1s = l[3]
            n2s = l[6]
            lb = [_rb16(v) for v in l]
            for kk in range(_K):
                idxk = idx_v[pl.ds(kk * npts + o, 16)]
                xk = _rb16(nbh_v[pl.ds((3 * kk + 0) * npts + o, 16)])
                yk = _rb16(nbh_v[pl.ds((3 * kk + 1) * npts + o, 16)])
                zk = _rb16(nbh_v[pl.ds((3 * kk + 2) * npts + o, 16)])
                p0 = xk * lb[0] + yk * lb[3] + zk * lb[6]
                p1 = xk * lb[1] + yk * lb[4] + zk * lb[7]
                p2 = xk * lb[2] + yk * lb[5] + zk * lb[8]
                ux = jnp.where(p0 >= 0.0, 4, 0)
                uy = jnp.where(p1 >= 0.0, 2, 0)
                uz = jnp.where(p2 >= 0.0, 1, 0)
                spatial = ux + uy + uz
                g0 = plsc.load_gather(n0, [idxk])
                g1 = plsc.load_gather(n1, [idxk])
                g2 = plsc.load_gather(n2, [idxk])
                cos = n0s * g0 + n1s * g1 + n2s * g2
                tval = _LOCAL_BINS * (cos + 1.0) / 2.0
                nid = tval.astype(jnp.int32)
                nid = jnp.minimum(jnp.maximum(nid, 0), _LOCAL_BINS - 1)
                binv = spatial * _LOCAL_BINS + nid
                flat = lane * _TOTAL_BINS + binv
                plsc.addupdate_scatter(hist_v, [flat], ones)
            pltpu.sync_copy(
                hist_v,
                out_hbm.at[pl.ds((wid * npts + o) * _TOTAL_BINS,
                                 16 * _TOTAL_BINS)])
            return carry

        lax.fori_loop(0, nchunks, chunk, 0)

    out = k(nbh_flat, idx_flat, nrm_flat, lrf_flat)
    return out.reshape(n, _TOTAL_BINS)


def kernel(points, batch):
    n = points.shape[0]
    nc, ns = _sc_workers()
    nw = nc * ns
    npts = n // nw
    nbh_idx = _knn_pallas(points)
    idx_sh = nbh_idx.T.reshape(_K, nw, npts).transpose(1, 0, 2).reshape(-1)
    cov_flat, nbh_flat = _cov_sc(points, idx_sh)
    covp = (cov_flat.reshape(nw, 6, npts).transpose(1, 0, 2)
            .reshape(6, n // 128, 128))
    lrfp = _eigh3_planes_pallas(covp)
    return _hist_sc(nbh_flat, idx_sh, lrfp)


def _kernel_old(points, batch):
    nbh_idx = _knn_pallas(points)
    nbh = points[nbh_idx]
    mean = jnp.mean(nbh, axis=1, keepdims=True)
    diffs = nbh - mean
    cov = jnp.einsum('nki,nkj->nij', diffs, diffs) / nbh.shape[1]
    _, lrfs = jnp.linalg.eigh(cov)
    normals = lrfs[:, :, 0]
    nbh_proj = jnp.einsum('nki,nij->nkj', nbh, lrfs)
    upper_x = (nbh_proj[:, :, 0] >= 0).astype(jnp.int32)
    upper_y = (nbh_proj[:, :, 1] >= 0).astype(jnp.int32)
    upper_z = (nbh_proj[:, :, 2] >= 0).astype(jnp.int32)
    spatial_id = (upper_x << 2) + (upper_y << 1) + upper_z
    cos = jnp.sum(normals[:, None, :] * normals[nbh_idx], axis=-1)
    normal_id = jnp.floor(_LOCAL_BINS * (cos + 1.0) / 2.0)
    normal_id = jnp.clip(normal_id, 0, _LOCAL_BINS - 1)
    bin_id = spatial_id.astype(jnp.float32) * _LOCAL_BINS + normal_id
    bin_idx = bin_id.astype(jnp.int32)
    rows = jnp.broadcast_to(jnp.arange(points.shape[0])[:, None], bin_idx.shape)
    shot = jnp.zeros((points.shape[0], _TOTAL_BINS), jnp.float32).at[rows, bin_idx].add(1.0)
    return shot


# speedup vs baseline: 14.2883x; 1.0003x over previous
import functools

import jax
import jax.numpy as jnp
from jax import lax
from jax.experimental import pallas as pl
from jax.experimental.pallas import tpu as pltpu
from jax.experimental.pallas import tpu_sc as plsc

_K = 5
_LOCAL_BINS = 10
_TOTAL_BINS = _LOCAL_BINS * 2 * 2 * 2

_INF = float("inf")
_BIG_I = 2**31 - 1


def _knn_body(rowb_ref, colb_ref, idx_ref, bval, bidx, *, ncol, blk_c):
    j = pl.program_id(1)

    @pl.when(j == 0)
    def _init():
        bval[...] = jnp.full(bval.shape, _INF, jnp.float32)
        bidx[...] = jnp.full(bidx.shape, _BIG_I, jnp.int32)

    def _rb(v):
        return v.astype(jnp.bfloat16).astype(jnp.float32)

    xr = _rb(rowb_ref[:, 0:1])
    yr = _rb(rowb_ref[:, 1:2])
    zr = _rb(rowb_ref[:, 2:3])
    pr = rowb_ref[:, 3:4]
    xc = _rb(colb_ref[0:1, :])
    yc = _rb(colb_ref[1:2, :])
    zc = _rb(colb_ref[2:3, :])
    pc = colb_ref[3:4, :]
    d = (pr + pc) - 2.0 * (xr * xc + yr * yc + zr * zc)

    cols = jax.lax.broadcasted_iota(jnp.int32, (1, blk_c), 1) + j * blk_c

    bv = []
    bi = []
    for _ in range(_K):
        m = jnp.min(d, axis=1, keepdims=True)
        am = jnp.min(jnp.where(d == m, cols, _BIG_I), axis=1, keepdims=True)
        bv.append(m)
        bi.append(am)
        d = jnp.where(cols == am, jnp.inf, d)

    cv = jnp.concatenate([bval[...]] + bv, axis=1)
    ci = jnp.concatenate([bidx[...]] + bi, axis=1)
    nv = []
    ni = []
    for _ in range(_K):
        m = jnp.min(cv, axis=1, keepdims=True)
        am = jnp.min(jnp.where(cv == m, ci, _BIG_I), axis=1, keepdims=True)
        nv.append(m)
        ni.append(am)
        cv = jnp.where(ci == am, jnp.inf, cv)
    bval[...] = jnp.concatenate(nv, axis=1)
    bidx[...] = jnp.concatenate(ni, axis=1)

    @pl.when(j == ncol - 1)
    def _flush():
        idx_ref[...] = bidx[...]


def _knn_pallas(points, interpret=False, blk_r=256, blk_c=8192):
    n = points.shape[0]
    blk_r = min(blk_r, n)
    blk_c = min(blk_c, n)
    nrow = n // blk_r
    ncol = n // blk_c
    pn = jnp.sum(points * points, axis=-1)
    rowb = jnp.concatenate([points, pn[:, None]], axis=1)
    colb = jnp.concatenate([points.T, pn[None, :]], axis=0)
    body = functools.partial(_knn_body, ncol=ncol, blk_c=blk_c)
    return pl.pallas_call(
        body,
        grid=(nrow, ncol),
        in_specs=[
            pl.BlockSpec((blk_r, 4), lambda i, j: (i, 0)),
            pl.BlockSpec((4, blk_c), lambda i, j: (0, j)),
        ],
        out_specs=pl.BlockSpec((blk_r, _K), lambda i, j: (i, 0)),
        out_shape=jax.ShapeDtypeStruct((n, _K), jnp.int32),
        scratch_shapes=[
            pltpu.VMEM((blk_r, _K), jnp.float32),
            pltpu.VMEM((blk_r, _K), jnp.int32),
        ],
        interpret=interpret,
    )(rowb, colb)


_SWEEPS = 6
_JACOBI_PAIRS = [(0, 2), (2, 1), (0, 1)]


def _eigh3_body(cin_ref, vout_ref):
    A = [[None] * 3 for _ in range(3)]
    A[0][0] = cin_ref[0]
    A[0][1] = A[1][0] = cin_ref[1]
    A[0][2] = A[2][0] = cin_ref[2]
    A[1][1] = cin_ref[3]
    A[1][2] = A[2][1] = cin_ref[4]
    A[2][2] = cin_ref[5]
    one = jnp.ones_like(A[0][0])
    zero = jnp.zeros_like(A[0][0])
    V = [[one if i == j else zero for j in range(3)] for i in range(3)]

    for _ in range(_SWEEPS):
        for (p, q) in _JACOBI_PAIRS:
            r = 3 - p - q
            wpp, wqq, wpq = A[p][p], A[q][q], A[p][q]
            tau = (wqq - wpp) / (2.0 * wpq)
            t = jnp.sign(tau) / (jnp.abs(tau) + jnp.sqrt(1.0 + tau * tau))
            c = jax.lax.rsqrt(1.0 + t * t)
            s = t * c
            c = jnp.where(wpq == 0.0, 1.0, c)
            s = jnp.where(wpq == 0.0, 0.0, s)
            cc, ss, cs = c * c, s * s, c * s
            apr, aqr = A[p][r], A[q][r]
            new_pp = cc * wpp - 2.0 * cs * wpq + ss * wqq
            new_qq = ss * wpp + 2.0 * cs * wpq + cc * wqq
            new_pq = (cc - ss) * wpq + cs * (wpp - wqq)
            new_pr = c * apr - s * aqr
            new_qr = s * apr + c * aqr
            A[p][p] = new_pp
            A[q][q] = new_qq
            A[p][q] = A[q][p] = new_pq
            A[p][r] = A[r][p] = new_pr
            A[q][r] = A[r][q] = new_qr
            for i in range(3):
                vip, viq = V[i][p], V[i][q]
                V[i][p] = c * vip - s * viq
                V[i][q] = s * vip + c * viq

    e0, e1, e2 = A[0][0], A[1][1], A[2][2]
    evs = [e0, e1, e2]
    m01 = jnp.where(e1 < e0, 1, 0)
    em01 = jnp.where(e1 < e0, e1, e0)
    first = jnp.where(e2 < em01, 2, m01)
    m21 = jnp.where(e1 > e2, 1, 2)
    em21 = jnp.where(e1 > e2, e1, e2)
    last = jnp.where(e0 > em21, 0, m21)
    middle = 3 - first - last
    for j, ordj in enumerate((first, middle, last)):
        for i in range(3):
            col = jnp.where(ordj == 0, V[i][0],
                            jnp.where(ordj == 1, V[i][1], V[i][2]))
            vout_ref[3 * i + j] = col


def _eigh3_planes_pallas(planes, interpret=False):
    _, rows, _ = planes.shape
    vp = pl.pallas_call(
        _eigh3_body,
        out_shape=jax.ShapeDtypeStruct((9, rows, 128), jnp.float32),
        interpret=interpret,
    )(planes)
    return vp.reshape(9, rows * 128)


def _eigh3_pallas(cov, interpret=False):
    n = cov.shape[0]
    planes = jnp.stack([
        cov[:, 0, 0], cov[:, 0, 1], cov[:, 0, 2],
        cov[:, 1, 1], cov[:, 1, 2], cov[:, 2, 2],
    ]).reshape(6, n // 128, 128)
    return _eigh3_planes_pallas(planes, interpret).T.reshape(n, 3, 3)


def _sc_workers():
    try:
        info = plsc.get_sparse_core_info()
        return info.num_cores, info.num_subcores
    except Exception:
        return 2, 16


def _rb16(v):
    u = plsc.bitcast(v, jnp.int32)
    r = (u + 0x7FFF + ((u >> 16) & 1)) & jnp.int32(-65536)
    return plsc.bitcast(r, jnp.float32)


def _cov_sc(points, idx_sh):
    n = points.shape[0]
    nc, ns = _sc_workers()
    nw = nc * ns
    npts = n // nw
    nchunks = npts // 16
    pts_flat = points.T.reshape(3 * n)
    mesh = plsc.VectorSubcoreMesh(core_axis_name="c", subcore_axis_name="s")

    @functools.partial(
        pl.kernel, mesh=mesh,
        compiler_params=pltpu.CompilerParams(needs_layout_passes=False),
        out_type=(jax.ShapeDtypeStruct((nw * 6 * npts,), jnp.float32),
                  jax.ShapeDtypeStruct((nw * 15 * npts,), jnp.float32)),
        scratch_types=[
            pltpu.VMEM((n,), jnp.float32),
            pltpu.VMEM((n,), jnp.float32),
            pltpu.VMEM((n,), jnp.float32),
            pltpu.VMEM((5 * npts,), jnp.int32),
            pltpu.VMEM((6 * npts,), jnp.float32),
            pltpu.VMEM((15 * npts,), jnp.float32),
        ],
    )
    def k(pts_hbm, idx_hbm, cov_hbm, nbh_hbm, px, py, pz, idx_v, cov_v, nbh_v):
        wid = lax.axis_index("s") * nc + lax.axis_index("c")
        pltpu.sync_copy(pts_hbm.at[pl.ds(0, n)], px)
        pltpu.sync_copy(pts_hbm.at[pl.ds(n, n)], py)
        pltpu.sync_copy(pts_hbm.at[pl.ds(2 * n, n)], pz)
        pltpu.sync_copy(idx_hbm.at[pl.ds(wid * 5 * npts, 5 * npts)], idx_v)

        def chunk(ci, carry):
            o = ci * 16
            xs, ys, zs = [], [], []
            for kk in range(_K):
                idxk = idx_v[pl.ds(kk * npts + o, 16)]
                xk = plsc.load_gather(px, [idxk])
                yk = plsc.load_gather(py, [idxk])
                zk = plsc.load_gather(pz, [idxk])
                nbh_v[pl.ds((3 * kk + 0) * npts + o, 16)] = xk
                nbh_v[pl.ds((3 * kk + 1) * npts + o, 16)] = yk
                nbh_v[pl.ds((3 * kk + 2) * npts + o, 16)] = zk
                xs.append(xk)
                ys.append(yk)
                zs.append(zk)
            mx = (((xs[0] + xs[4]) + xs[2]) + (xs[1] + xs[3])) * 0.2
            my = (((ys[0] + ys[4]) + ys[2]) + (ys[1] + ys[3])) * 0.2
            mz = (((zs[0] + zs[4]) + zs[2]) + (zs[1] + zs[3])) * 0.2
            dx = [_rb16(x - mx) for x in xs]
            dy = [_rb16(y - my) for y in ys]
            dz = [_rb16(z - mz) for z in zs]

            def dot5(a, b):
                p = [a[i] * b[i] for i in range(_K)]
                return (((p[0] + p[1]) + (p[2] + p[3])) + p[4]) * 0.2

            cov_v[pl.ds(0 * npts + o, 16)] = dot5(dx, dx)
            cov_v[pl.ds(1 * npts + o, 16)] = dot5(dx, dy)
            cov_v[pl.ds(2 * npts + o, 16)] = dot5(dx, dz)
            cov_v[pl.ds(3 * npts + o, 16)] = dot5(dy, dy)
            cov_v[pl.ds(4 * npts + o, 16)] = dot5(dy, dz)
            cov_v[pl.ds(5 * npts + o, 16)] = dot5(dz, dz)
            return carry

        lax.fori_loop(0, nchunks, chunk, 0)
        pltpu.sync_copy(cov_v, cov_hbm.at[pl.ds(wid * 6 * npts, 6 * npts)])
        pltpu.sync_copy(nbh_v, nbh_hbm.at[pl.ds(wid * 15 * npts, 15 * npts)])

    return k(pts_flat, idx_sh.reshape(-1))


def _hist_sc(nbh_flat, idx_flat, lrfp):
    nc, ns = _sc_workers()
    nw = nc * ns
    npts = nbh_flat.shape[0] // (15 * nw)
    n = nw * npts
    nchunks = npts // 16
    nrm_flat = jnp.concatenate([lrfp[0], lrfp[3], lrfp[6]], axis=0)
    lrf_flat = lrfp.reshape(9, nw, npts).transpose(1, 0, 2).reshape(-1)
    mesh = plsc.VectorSubcoreMesh(core_axis_name="c", subcore_axis_name="s")

    @functools.partial(
        pl.kernel, mesh=mesh,
        compiler_params=pltpu.CompilerParams(needs_layout_passes=False),
        out_type=jax.ShapeDtypeStruct((n * _TOTAL_BINS,), jnp.float32),
        scratch_types=[
            pltpu.VMEM((n,), jnp.float32),
            pltpu.VMEM((n,), jnp.float32),
            pltpu.VMEM((n,), jnp.float32),
            pltpu.VMEM((5 * npts,), jnp.int32),
            pltpu.VMEM((15 * npts,), jnp.float32),
            pltpu.VMEM((9 * npts,), jnp.float32),
            pltpu.VMEM((16 * _TOTAL_BINS,), jnp.float32),
        ],
    )
    def k(nbh_hbm, idx_hbm, nrm_hbm, lrf_hbm, out_hbm,
          n0, n1, n2, idx_v, nbh_v, lrf_v, hist_v):
        wid = lax.axis_index("s") * nc + lax.axis_index("c")
        pltpu.sync_copy(nrm_hbm.at[pl.ds(0, n)], n0)
        pltpu.sync_copy(nrm_hbm.at[pl.ds(n, n)], n1)
        pltpu.sync_copy(nrm_hbm.at[pl.ds(2 * n, n)], n2)
        pltpu.sync_copy(idx_hbm.at[pl.ds(wid * 5 * npts, 5 * npts)], idx_v)
        pltpu.sync_copy(nbh_hbm.at[pl.ds(wid * 15 * npts, 15 * npts)], nbh_v)
        pltpu.sync_copy(lrf_hbm.at[pl.ds(wid * 9 * npts, 9 * npts)], lrf_v)

        zeros = jnp.zeros((16,), jnp.float32)
        ones = jnp.ones((16,), jnp.float32)
        lane = lax.iota(jnp.int32, 16)

        def chunk(ci, carry):
            o = ci * 16
            for b in range(_TOTAL_BINS):
                hist_v[pl.ds(b * 16, 16)] = zeros
            l = [lrf_v[pl.ds(p * npts + o, 16)] for p in range(9)]
            n0s = l[0]
            n1s = l[3]
            n2s = l[6]
            lb = [_rb16(v) for v in l]
            for kk in range(_K):
                idxk = idx_v[pl.ds(kk * npts + o, 16)]
                xk = _rb16(nbh_v[pl.ds((3 * kk + 0) * npts + o, 16)])
                yk = _rb16(nbh_v[pl.ds((3 * kk + 1) * npts + o, 16)])
                zk = _rb16(nbh_v[pl.ds((3 * kk + 2) * npts + o, 16)])
                p0 = xk * lb[0] + yk * lb[3] + zk * lb[6]
                p1 = xk * lb[1] + yk * lb[4] + zk * lb[7]
                p2 = xk * lb[2] + yk * lb[5] + zk * lb[8]
                ux = jnp.where(p0 >= 0.0, 4, 0)
                uy = jnp.where(p1 >= 0.0, 2, 0)
                uz = jnp.where(p2 >= 0.0, 1, 0)
                spatial = ux + uy + uz
                g0 = plsc.load_gather(n0, [idxk])
                g1 = plsc.load_gather(n1, [idxk])
                g2 = plsc.load_gather(n2, [idxk])
                cos = n0s * g0 + n1s * g1 + n2s * g2
                tval = _LOCAL_BINS * (cos + 1.0) / 2.0
                nid = tval.astype(jnp.int32)
                nid = jnp.minimum(jnp.maximum(nid, 0), _LOCAL_BINS - 1)
                binv = spatial * _LOCAL_BINS + nid
                flat = lane * _TOTAL_BINS + binv
                plsc.addupdate_scatter(hist_v, [flat], ones)
            pltpu.sync_copy(
                hist_v,
                out_hbm.at[pl.ds((wid * npts + o) * _TOTAL_BINS,
                                 16 * _TOTAL_BINS)])
            return carry

        lax.fori_loop(0, nchunks, chunk, 0)

    out = k(nbh_flat, idx_flat, nrm_flat, lrf_flat)
    return out.reshape(n, _TOTAL_BINS)


def kernel(points, batch):
    n = points.shape[0]
    nc, ns = _sc_workers()
    nw = nc * ns
    npts = n // nw
    nbh_idx = _knn_pallas(points)
    idx_sh = nbh_idx.T.reshape(_K, nw, npts).transpose(1, 0, 2).reshape(-1)
    cov_flat, nbh_flat = _cov_sc(points, idx_sh)
    covp = (cov_flat.reshape(nw, 6, npts).transpose(1, 0, 2)
            .reshape(6, n // 128, 128))
    lrfp = _eigh3_planes_pallas(covp)
    return _hist_sc(nbh_flat, idx_sh, lrfp)
